# baseline (speedup 1.0000x reference)
import functools

import jax
import jax.numpy as jnp
from jax import lax
from jax.experimental import pallas as pl
from jax.experimental.pallas import tpu as pltpu
from jax.experimental.pallas import tpu_sc as plsc

_B, _C, _H, _W = 32, 192, 56, 56
_HW = _H * _W
_NG = 3
_EPS = 1e-10

_NC, _NS, _L = 2, 16, 16
_NW = _NC * _NS
_RPW = (_B * _C) // _NW
_CH = 8
_NCH = _RPW // _CH
_VPR = _HW // _L


def _stats_kernel(x_ref, w1_ref, b1_ref, w2_ref, b2_ref, wd1_ref, bd1_ref,
                  wd2_ref, bd2_ref, gum_ref, scale_ref, *, bblk):
    n = jnp.float32(_HW)
    for bi in range(bblk):
        xb = x_ref[bi]
        x2 = xb * xb
        x3 = x2 * xb
        x4 = x2 * x2
        s1 = jnp.sum(xb, axis=1, keepdims=True)
        s2 = jnp.sum(x2, axis=1, keepdims=True)
        s3 = jnp.sum(x3, axis=1, keepdims=True)
        s4 = jnp.sum(x4, axis=1, keepdims=True)

        mu = s1 / n
        e2 = s2 / n
        e3 = s3 / n
        e4 = s4 / n
        var0 = e2 - mu * mu
        m3c = e3 - 3.0 * mu * e2 + 2.0 * mu * mu * mu
        m4c = (e4 - 4.0 * mu * e3 + 6.0 * mu * mu * e2
               - 3.0 * (mu * mu) * (mu * mu))

        std = jnp.sqrt(var0)
        y2 = jnp.sqrt(var0 * (n / (n - 1.0)))
        skew = m3c / (std * std * std)
        kur = m4c / (var0 * var0)

        t = jnp.maximum(jnp.dot(w1_ref[...], mu,
                                preferred_element_type=jnp.float32)
                        + b1_ref[...][:, None], 0.0)
        logits = (jnp.dot(w2_ref[...], t, preferred_element_type=jnp.float32)
                  + b2_ref[...][:, None])

        g3 = gum_ref[bi]
        gsamp = -jnp.log(_EPS - jnp.log(g3 + _EPS))
        a = jnp.log(logits) + gsamp

        m = jnp.max(a)
        e = jnp.exp(a - m)
        sm = e / jnp.sum(e)

        s0 = sm[0, 0]
        s1g = sm[1, 0]
        s2g = sm[2, 0]
        best = s0
        idx = jnp.int32(0)
        c1 = (s1g > best) | (jnp.isnan(s1g) & ~jnp.isnan(best))
        idx = jnp.where(c1, jnp.int32(1), idx)
        best = jnp.where(c1, s1g, best)
        c2 = (s2g > best) | (jnp.isnan(s2g) & ~jnp.isnan(best))
        idx = jnp.where(c2, jnp.int32(2), idx)

        com = jnp.where(idx == 0, y2, jnp.where(idx == 1, skew, kur))

        d1 = jnp.maximum(jnp.dot(wd1_ref[...], com,
                                 preferred_element_type=jnp.float32)
                         + bd1_ref[...][:, None], 0.0)
        scale = jax.nn.sigmoid(jnp.dot(wd2_ref[...], d1,
                                       preferred_element_type=jnp.float32)
                               + bd2_ref[...][:, None])
        scale_ref[0, bi, :] = scale[:, 0]


def _compute_scales(x3d, W1, b1, W2, b2, Wd1, bd1, Wd2, bd2, gum):
    bblk = 4
    full = lambda i: (0, 0)
    scale = pl.pallas_call(
        functools.partial(_stats_kernel, bblk=bblk),
        grid=(_B // bblk,),
        in_specs=[
            pl.BlockSpec((bblk, _C, _HW), lambda i: (i, 0, 0)),
            pl.BlockSpec(W1.shape, full),
            pl.BlockSpec(b1.shape, lambda i: (0,)),
            pl.BlockSpec(W2.shape, full),
            pl.BlockSpec(b2.shape, lambda i: (0,)),
            pl.BlockSpec(Wd1.shape, full),
            pl.BlockSpec(bd1.shape, lambda i: (0,)),
            pl.BlockSpec(Wd2.shape, full),
            pl.BlockSpec(bd2.shape, lambda i: (0,)),
            pl.BlockSpec((bblk, _NG, 1), lambda i: (i, 0, 0)),
        ],
        out_specs=pl.BlockSpec((1, bblk, _C), lambda i: (i, 0, 0)),
        out_shape=jax.ShapeDtypeStruct((_B // bblk, bblk, _C), jnp.float32),
        compiler_params=pltpu.CompilerParams(
            dimension_semantics=("arbitrary",),
        ),
    )(x3d, W1, b1, W2, b2, Wd1, bd1, Wd2, bd2, gum)
    return scale.reshape(_B, _C)


_CHF = _CH * _HW
_UNR = 7
_NIT = _VPR // _UNR


def _sc_scale_kernel(x_hbm, s_hbm, out_hbm, xbuf, obuf, sbuf, insem, outsem):
    wid = lax.axis_index("s") * _NC + lax.axis_index("c")
    base = wid * _RPW * _HW
    pltpu.sync_copy(s_hbm.at[pl.ds(wid * _C, _C)],
                    sbuf.at[pl.ds(0, _C)])

    def in_copy(ch, buf):
        return pltpu.make_async_copy(
            x_hbm.at[pl.ds(base + ch * _CHF, _CHF)], xbuf.at[buf],
            insem.at[buf])

    def out_copy(ch, buf):
        return pltpu.make_async_copy(
            obuf.at[buf], out_hbm.at[pl.ds(base + ch * _CHF, _CHF)],
            outsem.at[buf])

    def compute(ch, buf):
        for r in range(_CH):
            svec = sbuf[pl.ds(ch * _CH + r, _L)]
            sv = jnp.full((_L,), svec[0], jnp.float32)
            rb = r * _HW

            def body(i, _, buf=buf, rb=rb, sv=sv):
                off = rb + i * (_UNR * _L)
                for j in range(_UNR):
                    sl = pl.ds(off + j * _L, _L)
                    obuf[buf, sl] = xbuf[buf, sl] * sv
                return 0

            lax.fori_loop(0, _NIT, body, 0)

    in_copy(0, 0).start()
    in_copy(1, 1).start()

    def pair_body(k, _):
        ch0 = 2 * k
        ch1 = 2 * k + 1
        in_copy(ch0, 0).wait()
        compute(ch0, 0)
        out_copy(ch0, 0).start()
        in_copy(ch0 + 2, 0).start()
        out_copy(ch0, 0).wait()
        in_copy(ch1, 1).wait()
        compute(ch1, 1)
        out_copy(ch1, 1).start()
        in_copy(ch1 + 2, 1).start()
        out_copy(ch1, 1).wait()
        return 0

    lax.fori_loop(0, _NCH // 2 - 1, pair_body, 0)
    for ch, buf in ((_NCH - 2, 0), (_NCH - 1, 1)):
        in_copy(ch, buf).wait()
        compute(ch, buf)
        cp = out_copy(ch, buf)
        cp.start()
        cp.wait()


@jax.jit
def kernel(x, W1, b1, W2, b2, Wd1, bd1, Wd2, bd2, gumbel_u):
    b, c, h, w_ = x.shape
    x3d = x.reshape(b, c, h * w_)
    gum = gumbel_u.reshape(b, _NG, 1)

    scale = _compute_scales(x3d, W1, b1, W2, b2, Wd1, bd1, Wd2, bd2, gum)

    x1d = x.reshape(b * c * h * w_)
    mesh = plsc.VectorSubcoreMesh(core_axis_name="c", subcore_axis_name="s")
    sc_call = functools.partial(
        pl.kernel, mesh=mesh,
        out_type=jax.ShapeDtypeStruct((b * c * h * w_,), jnp.float32),
        scratch_types=[
            pltpu.VMEM((2, _CHF), jnp.float32),
            pltpu.VMEM((2, _CHF), jnp.float32),
            pltpu.VMEM((_C + _L,), jnp.float32),
            pltpu.SemaphoreType.DMA((2,)),
            pltpu.SemaphoreType.DMA((2,)),
        ],
        compiler_params=pltpu.CompilerParams(use_tc_tiling_on_sc=True),
    )(_sc_scale_kernel)
    out1d = sc_call(x1d, scale.reshape(b * c))
    return out1d.reshape(b, c, h, w_)


# speedup vs baseline: 3.9803x; 3.9803x over previous
import functools

import jax
import jax.numpy as jnp
from jax.experimental import pallas as pl
from jax.experimental.pallas import tpu as pltpu

_B, _C, _H, _W = 32, 192, 56, 56
_HW = _H * _W
_NG = 3
_EPS = 1e-10


def _moca_kernel(x_ref, w1_ref, b1_ref, w2_ref, b2_ref, wd1_ref, bd1_ref,
                 wd2_ref, bd2_ref, gum_ref, out_ref, *, bblk):
    for bi in range(bblk):
        _moca_one(x_ref, w1_ref, b1_ref, w2_ref, b2_ref, wd1_ref, bd1_ref,
                  wd2_ref, bd2_ref, gum_ref, out_ref, bi)


def _moca_one(x_ref, w1_ref, b1_ref, w2_ref, b2_ref, wd1_ref, bd1_ref,
              wd2_ref, bd2_ref, gum_ref, out_ref, bi):
    xb = x_ref[bi]
    n = jnp.float32(_HW)

    x2 = xb * xb
    x3 = x2 * xb
    x4 = x2 * x2
    s1 = jnp.sum(xb, axis=1, keepdims=True)
    s2 = jnp.sum(x2, axis=1, keepdims=True)
    s3 = jnp.sum(x3, axis=1, keepdims=True)
    s4 = jnp.sum(x4, axis=1, keepdims=True)

    mu = s1 / n
    e2 = s2 / n
    e3 = s3 / n
    e4 = s4 / n
    var0 = e2 - mu * mu
    m3c = e3 - 3.0 * mu * e2 + 2.0 * mu * mu * mu
    m4c = e4 - 4.0 * mu * e3 + 6.0 * mu * mu * e2 - 3.0 * (mu * mu) * (mu * mu)

    std = jnp.sqrt(var0)
    y2 = jnp.sqrt(var0 * (n / (n - 1.0)))
    skew = m3c / (std * std * std)
    kur = m4c / (var0 * var0)

    t = jnp.maximum(jnp.dot(w1_ref[...], mu,
                            preferred_element_type=jnp.float32)
                    + b1_ref[...][:, None], 0.0)
    logits = jnp.dot(w2_ref[...], t,
                     preferred_element_type=jnp.float32) + b2_ref[...][:, None]

    g3 = gum_ref[bi]
    gsamp = -jnp.log(_EPS - jnp.log(g3 + _EPS))
    a = jnp.log(logits) + gsamp

    m = jnp.max(a)
    e = jnp.exp(a - m)
    sm = e / jnp.sum(e)

    s0 = sm[0, 0]
    s1g = sm[1, 0]
    s2g = sm[2, 0]
    best = s0
    idx = jnp.int32(0)
    c1 = (s1g > best) | (jnp.isnan(s1g) & ~jnp.isnan(best))
    idx = jnp.where(c1, jnp.int32(1), idx)
    best = jnp.where(c1, s1g, best)
    c2 = (s2g > best) | (jnp.isnan(s2g) & ~jnp.isnan(best))
    idx = jnp.where(c2, jnp.int32(2), idx)

    com = jnp.where(idx == 0, y2, jnp.where(idx == 1, skew, kur))

    d1 = jnp.maximum(jnp.dot(wd1_ref[...], com,
                             preferred_element_type=jnp.float32)
                     + bd1_ref[...][:, None], 0.0)
    scale = jax.nn.sigmoid(jnp.dot(wd2_ref[...], d1,
                                   preferred_element_type=jnp.float32)
                           + bd2_ref[...][:, None])

    out_ref[bi] = xb * scale


@jax.jit
def kernel(x, W1, b1, W2, b2, Wd1, bd1, Wd2, bd2, gumbel_u):
    b, c, h, w_ = x.shape
    x3 = x.reshape(b, c, h * w_)
    gum = gumbel_u.reshape(b, _NG, 1)

    bblk = 4
    full = lambda i: (0, 0)
    out = pl.pallas_call(
        functools.partial(_moca_kernel, bblk=bblk),
        grid=(b // bblk,),
        in_specs=[
            pl.BlockSpec((bblk, c, h * w_), lambda i: (i, 0, 0)),
            pl.BlockSpec(W1.shape, full),
            pl.BlockSpec(b1.shape, lambda i: (0,)),
            pl.BlockSpec(W2.shape, full),
            pl.BlockSpec(b2.shape, lambda i: (0,)),
            pl.BlockSpec(Wd1.shape, full),
            pl.BlockSpec(bd1.shape, lambda i: (0,)),
            pl.BlockSpec(Wd2.shape, full),
            pl.BlockSpec(bd2.shape, lambda i: (0,)),
            pl.BlockSpec((bblk, _NG, 1), lambda i: (i, 0, 0)),
        ],
        out_specs=pl.BlockSpec((bblk, c, h * w_), lambda i: (i, 0, 0)),
        out_shape=jax.ShapeDtypeStruct((b, c, h * w_), x.dtype),
        compiler_params=pltpu.CompilerParams(
            dimension_semantics=("arbitrary",),
        ),
    )(x3, W1, b1, W2, b2, Wd1, bd1, Wd2, bd2, gum)
    return out.reshape(b, c, h, w_)
